# Initial kernel scaffold; baseline (speedup 1.0000x reference)
import jax
import jax.numpy as jnp
from jax.experimental import pallas as pl


def kernel(pts, cb0, cb1, cb2, cb3):
    raise NotImplementedError("write your pallas kernel here")


# speedup vs baseline: 11.8763x; 11.8763x over previous
import functools

import jax
import jax.numpy as jnp
from jax import lax
from jax.experimental import pallas as pl
from jax.experimental.pallas import tpu as pltpu
from jax.experimental.pallas import tpu_sc as plsc

_LODS = (16, 32, 64, 128)
_F = 16
_N = 1048576
_NC = 2
_NS = 16
_L = 16
_NW = _NC * _NS
_PER_W = _N // _NW
_CH = 128
_NCH = _PER_W // _CH


def _sc_body(xyz_hbm, cb0, cb1, cb2, cb3, out_hbm, pbuf, ibuf, gbuf, gsem):
    cid = lax.axis_index("c")
    sid = lax.axis_index("s")
    wid = sid * _NC + cid
    base_chunk = wid * _NCH
    cbs = (cb0, cb1, cb2, cb3)

    @pl.loop(0, _NCH)
    def _chunk(c):
        chunk = base_chunk + c
        pltpu.sync_copy(xyz_hbm.at[chunk], pbuf)
        for i in range(_CH // _L):
            s = pl.ds(i * _L, _L)
            x = pbuf[0, s]
            y = pbuf[1, s]
            z = pbuf[2, s]
            for l, res in enumerate(_LODS):
                r = jnp.float32(res - 1)
                xi = (x * r).astype(jnp.int32)
                yi = (y * r).astype(jnp.int32)
                zi = (z * r).astype(jnp.int32)
                ibuf[l, s] = xi + yi * res + zi * (res * res)
        cps = [
            pltpu.async_copy(cbs[l].at[ibuf.at[l]], gbuf.at[l], gsem)
            for l in range(4)
        ]
        for cp in cps:
            cp.wait()
        for j in range(_CH):
            gbuf[0, j, :] = (
                gbuf[0, j, :] + gbuf[1, j, :] + gbuf[2, j, :] + gbuf[3, j, :]
            )
        pltpu.sync_copy(gbuf.at[0], out_hbm.at[pl.ds(chunk * _CH, _CH)])


@jax.jit
def _dense_grid_sc(xyz, cb0, cb1, cb2, cb3):
    mesh = plsc.VectorSubcoreMesh(core_axis_name="c", subcore_axis_name="s")
    return pl.kernel(
        _sc_body,
        out_type=jax.ShapeDtypeStruct((_N, _F), jnp.float32),
        mesh=mesh,
        compiler_params=pltpu.CompilerParams(use_tc_tiling_on_sc=False),
        scratch_types=[
            pltpu.VMEM((3, _CH), jnp.float32),
            pltpu.VMEM((4, _CH), jnp.int32),
            pltpu.VMEM((4, _CH, _F), jnp.float32),
            pltpu.SemaphoreType.DMA,
        ],
    )(xyz, cb0, cb1, cb2, cb3)


def kernel(pts, cb0, cb1, cb2, cb3):
    xyz = pts.T.reshape(3, _N // _CH, _CH).transpose(1, 0, 2)
    return _dense_grid_sc(xyz, cb0, cb1, cb2, cb3)


# speedup vs baseline: 12.2871x; 1.0346x over previous
import functools

import jax
import jax.numpy as jnp
from jax import lax
from jax.experimental import pallas as pl
from jax.experimental.pallas import tpu as pltpu
from jax.experimental.pallas import tpu_sc as plsc

_LODS = (16, 32, 64, 128)
_F = 16
_N = 1048576
_NC = 2
_NS = 16
_L = 16
_NW = _NC * _NS
_PER_W = _N // _NW
_CH = 128
_NCH = _PER_W // _CH


def _sc_body(xyz_hbm, cb0, cb1, cb2, cb3, out_hbm, pbuf, ibuf, gbuf,
             psem, gsem, ssem):
    cid = lax.axis_index("c")
    sid = lax.axis_index("s")
    wid = sid * _NC + cid
    base_chunk = wid * _NCH
    cbs = (cb0, cb1, cb2, cb3)

    def pts_start(c, b):
        cn = jnp.minimum(c, _NCH - 1)
        pltpu.async_copy(xyz_hbm.at[base_chunk + cn], pbuf.at[b], psem.at[b])

    def pts_wait(b):
        pltpu.make_async_copy(
            xyz_hbm.at[base_chunk], pbuf.at[b], psem.at[b]).wait()

    def idx_compute(b):
        for i in range(_CH // _L):
            s = pl.ds(i * _L, _L)
            x = pbuf[b, 0, s]
            y = pbuf[b, 1, s]
            z = pbuf[b, 2, s]
            for l, res in enumerate(_LODS):
                r = jnp.float32(res - 1)
                xi = (x * r).astype(jnp.int32)
                yi = (y * r).astype(jnp.int32)
                zi = (z * r).astype(jnp.int32)
                ibuf[b, l, s] = xi + yi * res + zi * (res * res)

    def gathers_start(b):
        for l in range(4):
            pltpu.async_copy(cbs[l].at[ibuf.at[b, l]], gbuf.at[b, l],
                             gsem.at[b])

    def gathers_wait(b):
        for l in range(4):
            pltpu.make_async_copy(cbs[l].at[ibuf.at[b, l]], gbuf.at[b, l],
                                  gsem.at[b]).wait()

    def sum_rows(b):
        for j in range(_CH):
            gbuf[b, 0, j, :] = (
                gbuf[b, 0, j, :] + gbuf[b, 1, j, :]
                + gbuf[b, 2, j, :] + gbuf[b, 3, j, :]
            )

    def store_start(c, b):
        pltpu.async_copy(
            gbuf.at[b, 0], out_hbm.at[pl.ds((base_chunk + c) * _CH, _CH)],
            ssem.at[b])

    def store_wait(b):
        pltpu.make_async_copy(
            gbuf.at[b, 0], out_hbm.at[pl.ds(base_chunk * _CH, _CH)],
            ssem.at[b]).wait()

    pts_start(0, 0)
    pts_start(1, 1)
    pts_wait(0)
    idx_compute(0)
    pts_start(2, 0)
    gathers_start(0)
    pts_wait(1)
    idx_compute(1)
    pts_start(3, 1)
    gathers_wait(0)
    sum_rows(0)
    gathers_start(1)
    store_start(0, 0)

    @pl.loop(0, (_NCH - 2) // 2)
    def _steady(cc):
        for b in range(2):
            c = 2 + 2 * cc + b
            pts_wait(b)
            idx_compute(b)
            pts_start(c + 2, b)
            gathers_wait(1 - b)
            sum_rows(1 - b)
            store_wait(b)
            gathers_start(b)
            store_start(c - 1, 1 - b)

    gathers_wait(1)
    sum_rows(1)
    store_start(_NCH - 1, 1)
    store_wait(0)
    store_wait(1)
    pts_wait(0)
    pts_wait(1)


@jax.jit
def _dense_grid_sc(xyz, cb0, cb1, cb2, cb3):
    mesh = plsc.VectorSubcoreMesh(core_axis_name="c", subcore_axis_name="s")
    return pl.kernel(
        _sc_body,
        out_type=jax.ShapeDtypeStruct((_N, _F), jnp.float32),
        mesh=mesh,
        compiler_params=pltpu.CompilerParams(use_tc_tiling_on_sc=False),
        scratch_types=[
            pltpu.VMEM((2, 3, _CH), jnp.float32),
            pltpu.VMEM((2, 4, _CH), jnp.int32),
            pltpu.VMEM((2, 4, _CH, _F), jnp.float32),
            pltpu.SemaphoreType.DMA((2,)),
            pltpu.SemaphoreType.DMA((2,)),
            pltpu.SemaphoreType.DMA((2,)),
        ],
    )(xyz, cb0, cb1, cb2, cb3)


def kernel(pts, cb0, cb1, cb2, cb3):
    xyz = pts.T.reshape(3, _N // _CH, _CH).transpose(1, 0, 2)
    return _dense_grid_sc(xyz, cb0, cb1, cb2, cb3)


# speedup vs baseline: 14.6865x; 1.1953x over previous
import functools

import jax
import jax.numpy as jnp
from jax import lax
from jax.experimental import pallas as pl
from jax.experimental.pallas import tpu as pltpu
from jax.experimental.pallas import tpu_sc as plsc

_LODS = (16, 32, 64, 128)
_F = 16
_N = 1048576
_NC = 2
_NS = 16
_L = 16
_NW = _NC * _NS
_PER_W = _N // _NW
_CH = 128
_NCH = _PER_W // _CH
_NCHT = _N // _CH


def _sc_body(xyz_hbm, cb0, cb1, cb2, cb3, out_hbm,
             pbuf, ibuf, gbuf, sbuf, psem, gsem, ssem):
    cid = lax.axis_index("c")
    sid = lax.axis_index("s")
    wid = sid * _NC + cid
    base_chunk = wid * _NCH
    cbs = (cb0, cb1, cb2, cb3)
    riota = lax.iota(jnp.int32, _L)

    def pts_start(c, b):
        cn = jnp.minimum(c, _NCH - 1)
        pltpu.async_copy(xyz_hbm.at[base_chunk + cn], pbuf.at[b], psem.at[b])

    def pts_wait(b):
        pltpu.make_async_copy(
            xyz_hbm.at[base_chunk], pbuf.at[b], psem.at[b]).wait()

    def idx_compute(b):
        @pl.loop(0, _CH // _L)
        def _(i):
            s = pl.ds(i * _L, _L)
            x = pbuf[b, 0, s]
            y = pbuf[b, 1, s]
            z = pbuf[b, 2, s]
            for l, res in enumerate(_LODS):
                r = jnp.float32(res - 1)
                xi = (x * r).astype(jnp.int32)
                yi = (y * r).astype(jnp.int32)
                zi = (z * r).astype(jnp.int32)
                ibuf[b, l, s] = xi + yi * res + zi * (res * res)

    def gathers_start(b):
        for l in range(4):
            pltpu.async_copy(cbs[l].at[ibuf.at[b, l]], gbuf.at[b, l],
                             gsem.at[b])

    def gathers_wait(b):
        for l in range(4):
            pltpu.make_async_copy(cbs[l].at[ibuf.at[b, l]], gbuf.at[b, l],
                                  gsem.at[b]).wait()

    def tsum(b):
        @pl.loop(0, _CH // _L)
        def _(g):
            ridx = riota + g * _L
            for f in range(_F):
                fidx = jnp.full((_L,), f, jnp.int32)
                acc = plsc.load_gather(gbuf.at[b, 0], [ridx, fidx])
                for l in range(1, 4):
                    acc = acc + plsc.load_gather(gbuf.at[b, l], [ridx, fidx])
                sbuf[b, f // 8, f % 8, pl.ds(g * _L, _L)] = acc

    def store_start(c, b):
        for r in range(2):
            pltpu.async_copy(sbuf.at[b, r], out_hbm.at[r, base_chunk + c],
                             ssem.at[b])

    def store_wait(b):
        for r in range(2):
            pltpu.make_async_copy(sbuf.at[b, r], out_hbm.at[r, base_chunk],
                                  ssem.at[b]).wait()

    pts_start(0, 0)
    pts_start(1, 1)
    pts_wait(0)
    idx_compute(0)
    pts_start(2, 0)
    gathers_start(0)
    pts_wait(1)
    idx_compute(1)
    pts_start(3, 1)
    gathers_start(1)
    gathers_wait(0)
    tsum(0)
    store_start(0, 0)
    pts_wait(0)
    idx_compute(0)
    pts_start(4, 0)
    gathers_start(0)
    gathers_wait(1)
    tsum(1)
    store_start(1, 1)
    pts_wait(1)
    idx_compute(1)
    pts_start(5, 1)
    gathers_start(1)
    gathers_wait(0)
    store_wait(0)
    tsum(0)
    store_start(2, 0)

    @pl.loop(0, (_NCH - 4) // 2)
    def _steady(cc):
        for b in range(2):
            c = 4 + 2 * cc + b
            pts_wait(b)
            idx_compute(b)
            pts_start(c + 2, b)
            gathers_start(b)
            gathers_wait(1 - b)
            store_wait(1 - b)
            tsum(1 - b)
            store_start(c - 1, 1 - b)

    gathers_wait(1)
    store_wait(1)
    tsum(1)
    store_start(_NCH - 1, 1)
    store_wait(0)
    store_wait(1)
    pts_wait(0)
    pts_wait(1)


@jax.jit
def _dense_grid_sc(xyz, cb0, cb1, cb2, cb3):
    mesh = plsc.VectorSubcoreMesh(core_axis_name="c", subcore_axis_name="s")
    return pl.kernel(
        _sc_body,
        out_type=jax.ShapeDtypeStruct((2, _NCHT, 8, _CH), jnp.float32),
        mesh=mesh,
        compiler_params=pltpu.CompilerParams(use_tc_tiling_on_sc=False,
                                             needs_layout_passes=False),
        scratch_types=[
            pltpu.VMEM((2, 3, _CH), jnp.float32),
            pltpu.VMEM((2, 4, _CH), jnp.int32),
            pltpu.VMEM((2, 4, _CH, _F), jnp.float32),
            pltpu.VMEM((2, 2, 8, _CH), jnp.float32),
            pltpu.SemaphoreType.DMA((2,)),
            pltpu.SemaphoreType.DMA((2,)),
            pltpu.SemaphoreType.DMA((2,)),
        ],
    )(xyz, cb0, cb1, cb2, cb3)


def kernel(pts, cb0, cb1, cb2, cb3):
    xyz = pts.T.reshape(3, _NCHT, _CH).transpose(1, 0, 2)
    out4d = _dense_grid_sc(xyz, cb0, cb1, cb2, cb3)
    return out4d.transpose(1, 3, 0, 2).reshape(_N, _F)


# speedup vs baseline: 20.3293x; 1.3842x over previous
import functools

import jax
import jax.numpy as jnp
from jax import lax
from jax.experimental import pallas as pl
from jax.experimental.pallas import tpu as pltpu
from jax.experimental.pallas import tpu_sc as plsc

_LODS = (16, 32, 64, 128)
_F = 16
_N = 1048576
_NC = 2
_NS = 16
_L = 16
_NW = _NC * _NS
_PER_W = _N // _NW
_CH = 128
_NCH = _PER_W // _CH
_NCHT = _N // _CH


def _sc_body(xyz_hbm, cb0, cb1, cb2, cb3, out_hbm,
             pbuf, ibuf, abuf, sbuf, psem, gsem, ssem):
    cid = lax.axis_index("c")
    sid = lax.axis_index("s")
    wid = sid * _NC + cid
    base_chunk = wid * _NCH
    cbs = (cb0, cb1, cb2, cb3)
    riota = lax.iota(jnp.int32, _L)
    zeros = jnp.zeros((_L,), jnp.float32)

    def pts_start(c, b):
        cn = jnp.minimum(c, _NCH - 1)
        pltpu.async_copy(xyz_hbm.at[base_chunk + cn], pbuf.at[b], psem.at[b])

    def pts_wait(b):
        pltpu.make_async_copy(
            xyz_hbm.at[base_chunk], pbuf.at[b], psem.at[b]).wait()

    def idx_compute(b):
        @pl.loop(0, _CH // _L)
        def _(i):
            s = pl.ds(i * _L, _L)
            x = pbuf[b, 0, s]
            y = pbuf[b, 1, s]
            z = pbuf[b, 2, s]
            for l, res in enumerate(_LODS):
                r = jnp.float32(res - 1)
                xi = (x * r).astype(jnp.int32)
                yi = (y * r).astype(jnp.int32)
                zi = (z * r).astype(jnp.int32)
                ibuf[b, l, s] = xi + yi * res + zi * (res * res)

    def gathers_start(b):
        for l in range(4):
            pltpu.async_copy(cbs[l].at[ibuf.at[b, l]], abuf.at[b],
                             gsem.at[b], add=True)

    def gathers_wait(b):
        for l in range(4):
            pltpu.make_async_copy(cbs[l].at[ibuf.at[b, l]], abuf.at[b],
                                  gsem.at[b]).wait()

    def zero_abuf(b):
        @pl.loop(0, _CH // _L)
        def _(g):
            ridx = riota + g * _L
            for f in range(_F):
                fidx = jnp.full((_L,), f, jnp.int32)
                plsc.store_scatter(abuf.at[b], [ridx, fidx], zeros)

    def tsum(b):
        @pl.loop(0, _CH // _L)
        def _(g):
            ridx = riota + g * _L
            for f in range(_F):
                fidx = jnp.full((_L,), f, jnp.int32)
                v = plsc.load_gather(abuf.at[b], [ridx, fidx])
                plsc.store_scatter(abuf.at[b], [ridx, fidx], zeros)
                sbuf[b, f // 8, f % 8, pl.ds(g * _L, _L)] = v

    def store_start(c, b):
        for r in range(2):
            pltpu.async_copy(sbuf.at[b, r], out_hbm.at[r, base_chunk + c],
                             ssem.at[b])

    def store_wait(b):
        for r in range(2):
            pltpu.make_async_copy(sbuf.at[b, r], out_hbm.at[r, base_chunk],
                                  ssem.at[b]).wait()

    zero_abuf(0)
    zero_abuf(1)
    pts_start(0, 0)
    pts_start(1, 1)
    pts_wait(0)
    idx_compute(0)
    pts_start(2, 0)
    gathers_start(0)
    pts_wait(1)
    idx_compute(1)
    pts_start(3, 1)
    gathers_start(1)
    gathers_wait(0)
    tsum(0)
    store_start(0, 0)
    pts_wait(0)
    idx_compute(0)
    pts_start(4, 0)
    gathers_start(0)
    gathers_wait(1)
    tsum(1)
    store_start(1, 1)
    pts_wait(1)
    idx_compute(1)
    pts_start(5, 1)
    gathers_start(1)
    gathers_wait(0)
    store_wait(0)
    tsum(0)
    store_start(2, 0)

    @pl.loop(0, (_NCH - 4) // 2)
    def _steady(cc):
        for b in range(2):
            c = 4 + 2 * cc + b
            pts_wait(b)
            idx_compute(b)
            pts_start(c + 2, b)
            gathers_start(b)
            gathers_wait(1 - b)
            store_wait(1 - b)
            tsum(1 - b)
            store_start(c - 1, 1 - b)

    gathers_wait(1)
    store_wait(1)
    tsum(1)
    store_start(_NCH - 1, 1)
    store_wait(0)
    store_wait(1)
    pts_wait(0)
    pts_wait(1)


@jax.jit
def _dense_grid_sc(xyz, cb0, cb1, cb2, cb3):
    mesh = plsc.VectorSubcoreMesh(core_axis_name="c", subcore_axis_name="s")
    return pl.kernel(
        _sc_body,
        out_type=jax.ShapeDtypeStruct((2, _NCHT, 8, _CH), jnp.float32),
        mesh=mesh,
        compiler_params=pltpu.CompilerParams(use_tc_tiling_on_sc=False,
                                             needs_layout_passes=False),
        scratch_types=[
            pltpu.VMEM((2, 3, _CH), jnp.float32),
            pltpu.VMEM((2, 4, _CH), jnp.int32),
            pltpu.VMEM((2, _CH, _F), jnp.float32),
            pltpu.VMEM((2, 2, 8, _CH), jnp.float32),
            pltpu.SemaphoreType.DMA((2,)),
            pltpu.SemaphoreType.DMA((2,)),
            pltpu.SemaphoreType.DMA((2,)),
        ],
    )(xyz, cb0, cb1, cb2, cb3)


def kernel(pts, cb0, cb1, cb2, cb3):
    xyz = pts.T.reshape(3, _NCHT, _CH).transpose(1, 0, 2)
    out4d = _dense_grid_sc(xyz, cb0, cb1, cb2, cb3)
    return out4d.transpose(1, 3, 0, 2).reshape(_N, _F)
